# baseline (device time: 781990 ns/iter reference)
import jax
import jax.numpy as jnp
from jax import lax
from jax.experimental import pallas as pl
from jax.experimental.pallas import tpu as pltpu

MC = 128
ME = 64
NSLOTS = 4


def kernel(A, B):
    M, Kl = A.shape
    K2, N = B.shape
    assert Kl == K2
    CM = (M - 2 * ME) // MC
    assert M == 2 * ME + CM * MC

    def body(a_hbm, b_hbm, o_hbm, b_vmem, a_vmem, p_vmem, recv_vmem,
             a_edge, p_edge, recv_edge,
             b_sem, a_sems, a_edge_sem, store_sems, edge_store_sem,
             send_sems, recv_sems, edge_send_sem, edge_recv_sem):
        my_x = lax.axis_index("x")
        my_y = lax.axis_index("y")
        nbr = (1 - my_x, my_y)

        def row(c):
            return c * MC - (MC - ME)

        def a_load(c):
            return pltpu.make_async_copy(
                a_hbm.at[pl.ds(row(c), MC)],
                a_vmem.at[lax.rem(c, 2)],
                a_sems.at[lax.rem(c, 2)])

        def exchange(c):
            return pltpu.make_async_remote_copy(
                src_ref=p_vmem.at[lax.rem(c, 2)],
                dst_ref=recv_vmem.at[lax.rem(c, NSLOTS)],
                send_sem=send_sems.at[lax.rem(c, 2)],
                recv_sem=recv_sems.at[lax.rem(c, NSLOTS)],
                device_id=nbr,
                device_id_type=pl.DeviceIdType.MESH,
            )

        def edge_exchange():
            return pltpu.make_async_remote_copy(
                src_ref=p_edge,
                dst_ref=recv_edge,
                send_sem=edge_send_sem,
                recv_sem=edge_recv_sem,
                device_id=nbr,
                device_id_type=pl.DeviceIdType.MESH,
            )

        def store(c):
            return pltpu.make_async_copy(
                p_vmem.at[lax.rem(c, 2)],
                o_hbm.at[pl.ds(row(c), MC)],
                store_sems.at[lax.rem(c, 2)])

        def edge_store(r):
            return pltpu.make_async_copy(
                p_edge, o_hbm.at[pl.ds(r, ME)], edge_store_sem)

        def consume(c):
            exchange(c).wait_recv()
            exchange(c).wait_send()
            slot = lax.rem(c, 2)
            p_vmem[slot] = p_vmem[slot] + recv_vmem[lax.rem(c, NSLOTS)]
            store(c).start()

        def consume_edge(r):
            edge_exchange().wait_recv()
            edge_exchange().wait_send()
            p_edge[...] = p_edge[...] + recv_edge[...]
            edge_store(r).start()

        barrier = pltpu.get_barrier_semaphore()
        pl.semaphore_signal(
            barrier, inc=1, device_id=nbr,
            device_id_type=pl.DeviceIdType.MESH,
        )
        pl.semaphore_wait(barrier, 1)

        b_load = pltpu.make_async_copy(b_hbm, b_vmem, b_sem)
        b_load.start()
        a_edge0 = pltpu.make_async_copy(
            a_hbm.at[pl.ds(0, ME)], a_edge, a_edge_sem)
        a_edge0.start()
        a_load(1).start()
        b_load.wait()
        a_edge0.wait()

        p_edge[...] = jnp.dot(
            a_edge[...], b_vmem[...], preferred_element_type=jnp.float32)
        edge_exchange().start()

        def step(c, carry):
            a_load(c).wait()

            @pl.when(c >= 3)
            def _():
                store(c - 2).wait()

            p_vmem[lax.rem(c, 2)] = jnp.dot(
                a_vmem[lax.rem(c, 2)], b_vmem[...],
                preferred_element_type=jnp.float32)
            exchange(c).start()

            @pl.when(c + 1 <= CM)
            def _():
                a_load(c + 1).start()

            @pl.when(c == CM)
            def _():
                pltpu.make_async_copy(
                    a_hbm.at[pl.ds(M - ME, ME)], a_edge, a_edge_sem,
                ).start()

            @pl.when(c == 1)
            def _():
                consume_edge(0)

            @pl.when(c >= 2)
            def _():
                consume(c - 1)

            return carry

        lax.fori_loop(1, CM + 1, step, 0)

        pltpu.make_async_copy(
            a_hbm.at[pl.ds(M - ME, ME)], a_edge, a_edge_sem).wait()
        edge_store(0).wait()
        p_edge[...] = jnp.dot(
            a_edge[...], b_vmem[...], preferred_element_type=jnp.float32)
        edge_exchange().start()
        consume(CM)
        consume_edge(M - ME)
        store(CM - 1).wait()
        store(CM).wait()
        edge_store(M - ME).wait()

    return pl.pallas_call(
        body,
        out_shape=jax.ShapeDtypeStruct((M, N), jnp.float32),
        in_specs=[
            pl.BlockSpec(memory_space=pl.ANY),
            pl.BlockSpec(memory_space=pl.ANY),
        ],
        out_specs=pl.BlockSpec(memory_space=pl.ANY),
        scratch_shapes=[
            pltpu.VMEM((K2, N), jnp.float32),
            pltpu.VMEM((2, MC, Kl), jnp.float32),
            pltpu.VMEM((2, MC, N), jnp.float32),
            pltpu.VMEM((NSLOTS, MC, N), jnp.float32),
            pltpu.VMEM((ME, Kl), jnp.float32),
            pltpu.VMEM((ME, N), jnp.float32),
            pltpu.VMEM((ME, N), jnp.float32),
            pltpu.SemaphoreType.DMA,
            pltpu.SemaphoreType.DMA((2,)),
            pltpu.SemaphoreType.DMA,
            pltpu.SemaphoreType.DMA((2,)),
            pltpu.SemaphoreType.DMA,
            pltpu.SemaphoreType.DMA((2,)),
            pltpu.SemaphoreType.DMA((NSLOTS,)),
            pltpu.SemaphoreType.DMA,
            pltpu.SemaphoreType.DMA,
        ],
        compiler_params=pltpu.CompilerParams(
            collective_id=0,
            vmem_limit_bytes=62 * 1024 * 1024,
        ),
    )(A, B)
